# baseline (speedup 1.0000x reference)
import functools

import jax
import jax.numpy as jnp
from jax import lax
from jax.experimental import pallas as pl
from jax.experimental.pallas import tpu as pltpu
from jax.experimental.pallas import tpu_sc as plsc

_N = 10000
_H = 128
_E = 320000
_NC = 2
_NS = 16
_NW = _NC * _NS
_CH = 500
_ROWS_W = 20
_NB = 4
_NBLK = _ROWS_W // _NB
_NPAD = 10112
_RPT = _NPAD // _NS

_mesh = plsc.VectorSubcoreMesh(core_axis_name="c", subcore_axis_name="s")
_sc_params = pltpu.CompilerParams(use_tc_tiling_on_sc=False)

_ZPIECES = tuple((o, min(_CH, _RPT - o)) for o in range(0, _RPT, _CH))


def _zero_rows(ref, nrows, width):
    zv = jnp.zeros((16,), jnp.float32)

    @pl.loop(0, nrows)
    def _(r):
        @pl.loop(0, width, step=16)
        def _(k):
            ref[r, pl.ds(k, 16)] = zv


def _sc_agg_body(h_hbm, srcp_hbm, dstp_hbm, out_hbm,
                 acc, srcv0, dstv0, srcv1, dstv1,
                 rows0, rows1, sem0, sem1, semi):
    c = lax.axis_index("c")
    s = lax.axis_index("s")
    wid = c * _NS + s
    zv = jnp.zeros((32,), jnp.bfloat16)

    @pl.loop(0, _CH)
    def _(r):
        @pl.loop(0, _H, step=32)
        def _(k):
            rows0[r, pl.ds(k, 32)] = zv
    zb = s * _RPT
    zcps = [pltpu.async_copy(rows0.at[pl.ds(0, n)],
                             acc.at[pl.ds(zb + off, n)], semi)
            for off, n in _ZPIECES]
    for zcp in zcps:
        zcp.wait()
    plsc.subcore_barrier()

    idx_bufs = ((srcv0, dstv0), (srcv1, dstv1))
    base0 = wid * _ROWS_W
    pltpu.sync_copy(srcp_hbm.at[pl.ds(base0, _NB)], srcv0)
    pltpu.sync_copy(dstp_hbm.at[pl.ds(base0, _NB)], dstv0)
    for b in range(_NBLK):
        srcv, dstv = idx_bufs[b % 2]
        nsrcv, ndstv = idx_bufs[(b + 1) % 2]
        if b + 1 < _NBLK:
            nbase = base0 + (b + 1) * _NB
            cpsi = pltpu.async_copy(srcp_hbm.at[pl.ds(nbase, _NB)], nsrcv, semi)
            cpdi = pltpu.async_copy(dstp_hbm.at[pl.ds(nbase, _NB)], ndstv, semi)
        pltpu.async_copy(h_hbm.at[srcv.at[0]], rows0, sem0)

        @pl.loop(0, _NB, step=2)
        def _(j):
            cp1 = pltpu.async_copy(h_hbm.at[srcv.at[j + 1]], rows1, sem1)
            pltpu.make_async_copy(h_hbm.at[srcv.at[j]], rows0, sem0).wait()
            pltpu.sync_copy(rows0, acc.at[dstv.at[j]], add=True)

            @pl.when(j + 2 < _NB)
            def _():
                pltpu.async_copy(h_hbm.at[srcv.at[j + 2]], rows0, sem0)

            cp1.wait()
            pltpu.sync_copy(rows1, acc.at[dstv.at[j + 1]], add=True)

        if b + 1 < _NBLK:
            cpsi.wait()
            cpdi.wait()

    plsc.subcore_barrier()
    pltpu.sync_copy(acc.at[pl.ds(zb, _RPT)], out_hbm.at[c, pl.ds(zb, _RPT)])


_sc_agg = pl.kernel(
    _sc_agg_body,
    out_type=jax.ShapeDtypeStruct((_NC, _NPAD, _H), jnp.bfloat16),
    mesh=_mesh,
    scratch_types=[
        pltpu.VMEM_SHARED((_NPAD, _H), jnp.bfloat16),
        pltpu.VMEM((_NB, _CH), jnp.int32),
        pltpu.VMEM((_NB, _CH), jnp.int32),
        pltpu.VMEM((_NB, _CH), jnp.int32),
        pltpu.VMEM((_NB, _CH), jnp.int32),
        pltpu.VMEM((_CH, _H), jnp.bfloat16),
        pltpu.VMEM((_CH, _H), jnp.bfloat16),
        pltpu.SemaphoreType.DMA,
        pltpu.SemaphoreType.DMA,
        pltpu.SemaphoreType.DMA,
    ],
    compiler_params=_sc_params,
)


def _sc_cnt_body(dstp_hbm, cnt_hbm, cnt_acc, dstv, onesv, sem0):
    c = lax.axis_index("c")
    s = lax.axis_index("s")
    wid = c * _NS + s
    _zero_rows(onesv, _CH, 16)
    zb = s * _RPT
    for off, n in _ZPIECES:
        pltpu.sync_copy(onesv.at[pl.ds(0, n)], cnt_acc.at[pl.ds(zb + off, n)])
    ov = jnp.full((16,), 1.0, jnp.float32)

    @pl.loop(0, _CH)
    def _(r):
        onesv[r, :] = ov

    plsc.subcore_barrier()

    @pl.loop(0, _NBLK)
    def _(b):
        pltpu.sync_copy(
            dstp_hbm.at[pl.ds(wid * _ROWS_W + b * _NB, _NB)], dstv)

        @pl.loop(0, _NB)
        def _(j):
            pltpu.sync_copy(onesv, cnt_acc.at[dstv.at[j]], add=True)

    plsc.subcore_barrier()
    pltpu.sync_copy(cnt_acc.at[pl.ds(zb, _RPT)], cnt_hbm.at[c, pl.ds(zb, _RPT)])


_sc_cnt = pl.kernel(
    _sc_cnt_body,
    out_type=jax.ShapeDtypeStruct((_NC, _NPAD, 16), jnp.float32),
    mesh=_mesh,
    scratch_types=[
        pltpu.VMEM_SHARED((_NPAD, 16), jnp.float32),
        pltpu.VMEM((_NB, _CH), jnp.int32),
        pltpu.VMEM((_CH, 16), jnp.float32),
        pltpu.SemaphoreType.DMA,
    ],
    compiler_params=_sc_params,
)


def _ln_relu(y, g, be):
    m = jnp.mean(y, axis=-1, keepdims=True)
    v = jnp.mean((y - m) * (y - m), axis=-1, keepdims=True)
    return jax.nn.relu((y - m) * lax.rsqrt(v + 1e-5) * g + be)


def _in_body(x_ref, w_ref, b_ref, g_ref, be_ref, wr_ref, blr_ref,
             hb_ref, hr_ref):
    y = jnp.dot(x_ref[...], w_ref[...], preferred_element_type=jnp.float32)
    h = _ln_relu(y + b_ref[...], g_ref[...], be_ref[...])
    hb_ref[...] = h.astype(jnp.bfloat16)
    hr_ref[...] = jnp.dot(h, wr_ref[...],
                          preferred_element_type=jnp.float32) + blr_ref[...]


def _combine(p_ref, inv, hr_ref, wl_ref, g_ref, be_ref):
    psum = (p_ref[0, :_N, :].astype(jnp.float32)
            + p_ref[1, :_N, :].astype(jnp.float32))
    agg = psum * inv
    out = jnp.dot(agg, wl_ref[...], preferred_element_type=jnp.float32)
    out = out + hr_ref[...]
    nrm = jnp.sqrt(jnp.sum(out * out, axis=-1, keepdims=True))
    out = out / jnp.maximum(nrm, 1e-12)
    return _ln_relu(out, g_ref[...], be_ref[...])


def _comb0_body(p_ref, cp_ref, hr_ref, wl_ref, g_ref, be_ref,
                wr_ref, blr_ref, h_ref, hb_ref, inv_ref, hrn_ref):
    cnt = cp_ref[0, :_N, :] + cp_ref[1, :_N, :]
    inv16 = 1.0 / jnp.maximum(cnt, 1.0)
    inv_ref[...] = inv16
    h = _combine(p_ref, inv16[:, 0:1], hr_ref, wl_ref, g_ref, be_ref)
    h_ref[...] = h
    hb_ref[...] = h.astype(jnp.bfloat16)
    hrn_ref[...] = jnp.dot(h, wr_ref[...],
                           preferred_element_type=jnp.float32) + blr_ref[...]


def _comb_mid_body(p_ref, inv_ref, hr_ref, wl_ref, g_ref, be_ref,
                   wr_ref, blr_ref, h_ref, hb_ref, hrn_ref):
    h = _combine(p_ref, inv_ref[:, 0:1], hr_ref, wl_ref, g_ref, be_ref)
    h_ref[...] = h
    hb_ref[...] = h.astype(jnp.bfloat16)
    hrn_ref[...] = jnp.dot(h, wr_ref[...],
                           preferred_element_type=jnp.float32) + blr_ref[...]


def _comb_last_head_body(p_ref, inv_ref, hr_ref, wl_ref, g_ref, be_ref,
                         h1_ref, h2_ref, wjk_ref, bjk_ref, gjk_ref, bejk_ref,
                         wp1_ref, bp1_ref, gp_ref, bep_ref, wp2_ref, bp2_ref,
                         o_ref):
    h3 = _combine(p_ref, inv_ref[:, 0:1], hr_ref, wl_ref, g_ref, be_ref)
    y = jnp.dot(h1_ref[...], wjk_ref[0:128, :], preferred_element_type=jnp.float32)
    y = y + jnp.dot(h2_ref[...], wjk_ref[128:256, :], preferred_element_type=jnp.float32)
    y = y + jnp.dot(h3, wjk_ref[256:384, :], preferred_element_type=jnp.float32)
    h = _ln_relu(y + bjk_ref[...], gjk_ref[...], bejk_ref[...])
    y2 = jnp.dot(h, wp1_ref[...], preferred_element_type=jnp.float32) + bp1_ref[...]
    hp = _ln_relu(y2, gp_ref[...], bep_ref[...])
    o_ref[...] = jnp.dot(hp, wp2_ref[...], preferred_element_type=jnp.float32) + bp2_ref[...]


def _tc(body, *out_specs):
    if not out_specs:
        out_specs = ((_H, jnp.float32),)
    shapes = tuple(jax.ShapeDtypeStruct((_N, c), dt) for c, dt in out_specs)
    return pl.pallas_call(
        body, out_shape=shapes if len(shapes) > 1 else shapes[0])


def kernel(x, edge_index, W_in, b_in, g_in, be_in, W_l0, b_l0, W_r0, g_0, be_0,
           W_l1, b_l1, W_r1, g_1, be_1, W_l2, b_l2, W_r2, g_2, be_2,
           W_jk, b_jk, g_jk, be_jk, W_p1, b_p1, g_p, be_p, W_p2, b_p2):
    srcp = edge_index[0].reshape(_E // _CH, _CH)
    dstp = edge_index[1].reshape(_E // _CH, _CH)

    cp = _sc_cnt(dstp)
    h0b, hr0 = _tc(_in_body, (_H, jnp.bfloat16), (_H, jnp.float32))(
        x, W_in, b_in, g_in, be_in, W_r0, b_l0)

    p0 = _sc_agg(h0b, srcp, dstp)
    h1, h1b, inv16, hr1 = _tc(
        _comb0_body, (_H, jnp.float32), (_H, jnp.bfloat16),
        (16, jnp.float32), (_H, jnp.float32))(
        p0, cp, hr0, W_l0, g_0, be_0, W_r1, b_l1)

    p1 = _sc_agg(h1b, srcp, dstp)
    h2, h2b, hr2 = _tc(
        _comb_mid_body, (_H, jnp.float32), (_H, jnp.bfloat16),
        (_H, jnp.float32))(
        p1, inv16, hr1, W_l1, g_1, be_1, W_r2, b_l2)

    p2 = _sc_agg(h2b, srcp, dstp)
    return _tc(_comb_last_head_body)(
        p2, inv16, hr2, W_l2, g_2, be_2, h1, h2,
        W_jk, b_jk, g_jk, be_jk, W_p1, b_p1, g_p, be_p, W_p2, b_p2)


# speedup vs baseline: 1.0329x; 1.0329x over previous
import functools

import jax
import jax.numpy as jnp
from jax import lax
from jax.experimental import pallas as pl
from jax.experimental.pallas import tpu as pltpu
from jax.experimental.pallas import tpu_sc as plsc

_N = 10000
_H = 128
_E = 320000
_NC = 2
_NS = 16
_NW = _NC * _NS
_CH = 250
_ROWS_W = 40
_NB = 8
_NBLK = _ROWS_W // _NB
_NPAD = 10112
_RPT = _NPAD // _NS

_mesh = plsc.VectorSubcoreMesh(core_axis_name="c", subcore_axis_name="s")
_sc_params = pltpu.CompilerParams(use_tc_tiling_on_sc=False)

_ZPIECES = tuple((o, min(_CH, _RPT - o)) for o in range(0, _RPT, _CH))


def _zero_rows(ref, nrows, width):
    zv = jnp.zeros((16,), jnp.float32)

    @pl.loop(0, nrows)
    def _(r):
        @pl.loop(0, width, step=16)
        def _(k):
            ref[r, pl.ds(k, 16)] = zv


def _sc_agg_body(h_hbm, srcp_hbm, dstp_hbm, out_hbm,
                 acc, srcv0, dstv0, srcv1, dstv1,
                 rows0, rows1, sem0, sem1, semi):
    c = lax.axis_index("c")
    s = lax.axis_index("s")
    wid = c * _NS + s
    zv = jnp.zeros((32,), jnp.bfloat16)

    @pl.loop(0, _CH)
    def _(r):
        @pl.loop(0, _H, step=32)
        def _(k):
            rows0[r, pl.ds(k, 32)] = zv
    zb = s * _RPT
    zcps = [pltpu.async_copy(rows0.at[pl.ds(0, n)],
                             acc.at[pl.ds(zb + off, n)], semi)
            for off, n in _ZPIECES]
    for zcp in zcps:
        zcp.wait()
    plsc.subcore_barrier()

    idx_bufs = ((srcv0, dstv0), (srcv1, dstv1))
    base0 = wid * _ROWS_W
    pltpu.sync_copy(srcp_hbm.at[pl.ds(base0, _NB)], srcv0)
    pltpu.sync_copy(dstp_hbm.at[pl.ds(base0, _NB)], dstv0)
    for b in range(_NBLK):
        srcv, dstv = idx_bufs[b % 2]
        nsrcv, ndstv = idx_bufs[(b + 1) % 2]
        if b + 1 < _NBLK:
            nbase = base0 + (b + 1) * _NB
            cpsi = pltpu.async_copy(srcp_hbm.at[pl.ds(nbase, _NB)], nsrcv, semi)
            cpdi = pltpu.async_copy(dstp_hbm.at[pl.ds(nbase, _NB)], ndstv, semi)
        pltpu.async_copy(h_hbm.at[srcv.at[0]], rows0, sem0)

        @pl.loop(0, _NB, step=2)
        def _(j):
            cp1 = pltpu.async_copy(h_hbm.at[srcv.at[j + 1]], rows1, sem1)
            pltpu.make_async_copy(h_hbm.at[srcv.at[j]], rows0, sem0).wait()
            pltpu.sync_copy(rows0, acc.at[dstv.at[j]], add=True)

            @pl.when(j + 2 < _NB)
            def _():
                pltpu.async_copy(h_hbm.at[srcv.at[j + 2]], rows0, sem0)

            cp1.wait()
            pltpu.sync_copy(rows1, acc.at[dstv.at[j + 1]], add=True)

        if b + 1 < _NBLK:
            cpsi.wait()
            cpdi.wait()

    plsc.subcore_barrier()
    pltpu.sync_copy(acc.at[pl.ds(zb, _RPT)], out_hbm.at[c, pl.ds(zb, _RPT)])


_sc_agg = pl.kernel(
    _sc_agg_body,
    out_type=jax.ShapeDtypeStruct((_NC, _NPAD, _H), jnp.bfloat16),
    mesh=_mesh,
    scratch_types=[
        pltpu.VMEM_SHARED((_NPAD, _H), jnp.bfloat16),
        pltpu.VMEM((_NB, _CH), jnp.int32),
        pltpu.VMEM((_NB, _CH), jnp.int32),
        pltpu.VMEM((_NB, _CH), jnp.int32),
        pltpu.VMEM((_NB, _CH), jnp.int32),
        pltpu.VMEM((_CH, _H), jnp.bfloat16),
        pltpu.VMEM((_CH, _H), jnp.bfloat16),
        pltpu.SemaphoreType.DMA,
        pltpu.SemaphoreType.DMA,
        pltpu.SemaphoreType.DMA,
    ],
    compiler_params=_sc_params,
)


def _sc_cnt_body(dstp_hbm, cnt_hbm, cnt_acc, dstv, onesv, sem0):
    c = lax.axis_index("c")
    s = lax.axis_index("s")
    wid = c * _NS + s
    _zero_rows(onesv, _CH, 16)
    zb = s * _RPT
    for off, n in _ZPIECES:
        pltpu.sync_copy(onesv.at[pl.ds(0, n)], cnt_acc.at[pl.ds(zb + off, n)])
    ov = jnp.full((16,), 1.0, jnp.float32)

    @pl.loop(0, _CH)
    def _(r):
        onesv[r, :] = ov

    plsc.subcore_barrier()

    @pl.loop(0, _NBLK)
    def _(b):
        pltpu.sync_copy(
            dstp_hbm.at[pl.ds(wid * _ROWS_W + b * _NB, _NB)], dstv)

        @pl.loop(0, _NB)
        def _(j):
            pltpu.sync_copy(onesv, cnt_acc.at[dstv.at[j]], add=True)

    plsc.subcore_barrier()
    pltpu.sync_copy(cnt_acc.at[pl.ds(zb, _RPT)], cnt_hbm.at[c, pl.ds(zb, _RPT)])


_sc_cnt = pl.kernel(
    _sc_cnt_body,
    out_type=jax.ShapeDtypeStruct((_NC, _NPAD, 16), jnp.float32),
    mesh=_mesh,
    scratch_types=[
        pltpu.VMEM_SHARED((_NPAD, 16), jnp.float32),
        pltpu.VMEM((_NB, _CH), jnp.int32),
        pltpu.VMEM((_CH, 16), jnp.float32),
        pltpu.SemaphoreType.DMA,
    ],
    compiler_params=_sc_params,
)


def _ln_relu(y, g, be):
    m = jnp.mean(y, axis=-1, keepdims=True)
    v = jnp.mean((y - m) * (y - m), axis=-1, keepdims=True)
    return jax.nn.relu((y - m) * lax.rsqrt(v + 1e-5) * g + be)


def _in_body(x_ref, w_ref, b_ref, g_ref, be_ref, wr_ref, blr_ref,
             hb_ref, hr_ref):
    y = jnp.dot(x_ref[...], w_ref[...], preferred_element_type=jnp.float32)
    h = _ln_relu(y + b_ref[...], g_ref[...], be_ref[...])
    hb_ref[...] = h.astype(jnp.bfloat16)
    hr_ref[...] = jnp.dot(h, wr_ref[...],
                          preferred_element_type=jnp.float32) + blr_ref[...]


def _combine(p_ref, inv, hr_ref, wl_ref, g_ref, be_ref):
    psum = (p_ref[0, :_N, :].astype(jnp.float32)
            + p_ref[1, :_N, :].astype(jnp.float32))
    agg = psum * inv
    out = jnp.dot(agg, wl_ref[...], preferred_element_type=jnp.float32)
    out = out + hr_ref[...]
    nrm = jnp.sqrt(jnp.sum(out * out, axis=-1, keepdims=True))
    out = out / jnp.maximum(nrm, 1e-12)
    return _ln_relu(out, g_ref[...], be_ref[...])


def _comb0_body(p_ref, cp_ref, hr_ref, wl_ref, g_ref, be_ref,
                wr_ref, blr_ref, h_ref, hb_ref, inv_ref, hrn_ref):
    cnt = cp_ref[0, :_N, :] + cp_ref[1, :_N, :]
    inv16 = 1.0 / jnp.maximum(cnt, 1.0)
    inv_ref[...] = inv16
    h = _combine(p_ref, inv16[:, 0:1], hr_ref, wl_ref, g_ref, be_ref)
    h_ref[...] = h
    hb_ref[...] = h.astype(jnp.bfloat16)
    hrn_ref[...] = jnp.dot(h, wr_ref[...],
                           preferred_element_type=jnp.float32) + blr_ref[...]


def _comb_mid_body(p_ref, inv_ref, hr_ref, wl_ref, g_ref, be_ref,
                   wr_ref, blr_ref, h_ref, hb_ref, hrn_ref):
    h = _combine(p_ref, inv_ref[:, 0:1], hr_ref, wl_ref, g_ref, be_ref)
    h_ref[...] = h
    hb_ref[...] = h.astype(jnp.bfloat16)
    hrn_ref[...] = jnp.dot(h, wr_ref[...],
                           preferred_element_type=jnp.float32) + blr_ref[...]


def _comb_last_head_body(p_ref, inv_ref, hr_ref, wl_ref, g_ref, be_ref,
                         h1_ref, h2_ref, wjk_ref, bjk_ref, gjk_ref, bejk_ref,
                         wp1_ref, bp1_ref, gp_ref, bep_ref, wp2_ref, bp2_ref,
                         o_ref):
    h3 = _combine(p_ref, inv_ref[:, 0:1], hr_ref, wl_ref, g_ref, be_ref)
    y = jnp.dot(h1_ref[...], wjk_ref[0:128, :], preferred_element_type=jnp.float32)
    y = y + jnp.dot(h2_ref[...], wjk_ref[128:256, :], preferred_element_type=jnp.float32)
    y = y + jnp.dot(h3, wjk_ref[256:384, :], preferred_element_type=jnp.float32)
    h = _ln_relu(y + bjk_ref[...], gjk_ref[...], bejk_ref[...])
    y2 = jnp.dot(h, wp1_ref[...], preferred_element_type=jnp.float32) + bp1_ref[...]
    hp = _ln_relu(y2, gp_ref[...], bep_ref[...])
    o_ref[...] = jnp.dot(hp, wp2_ref[...], preferred_element_type=jnp.float32) + bp2_ref[...]


def _tc(body, *out_specs):
    if not out_specs:
        out_specs = ((_H, jnp.float32),)
    shapes = tuple(jax.ShapeDtypeStruct((_N, c), dt) for c, dt in out_specs)
    return pl.pallas_call(
        body, out_shape=shapes if len(shapes) > 1 else shapes[0])


def kernel(x, edge_index, W_in, b_in, g_in, be_in, W_l0, b_l0, W_r0, g_0, be_0,
           W_l1, b_l1, W_r1, g_1, be_1, W_l2, b_l2, W_r2, g_2, be_2,
           W_jk, b_jk, g_jk, be_jk, W_p1, b_p1, g_p, be_p, W_p2, b_p2):
    srcp = edge_index[0].reshape(_E // _CH, _CH)
    dstp = edge_index[1].reshape(_E // _CH, _CH)

    cp = _sc_cnt(dstp)
    h0b, hr0 = _tc(_in_body, (_H, jnp.bfloat16), (_H, jnp.float32))(
        x, W_in, b_in, g_in, be_in, W_r0, b_l0)

    p0 = _sc_agg(h0b, srcp, dstp)
    h1, h1b, inv16, hr1 = _tc(
        _comb0_body, (_H, jnp.float32), (_H, jnp.bfloat16),
        (16, jnp.float32), (_H, jnp.float32))(
        p0, cp, hr0, W_l0, g_0, be_0, W_r1, b_l1)

    p1 = _sc_agg(h1b, srcp, dstp)
    h2, h2b, hr2 = _tc(
        _comb_mid_body, (_H, jnp.float32), (_H, jnp.bfloat16),
        (_H, jnp.float32))(
        p1, inv16, hr1, W_l1, g_1, be_1, W_r2, b_l2)

    p2 = _sc_agg(h2b, srcp, dstp)
    return _tc(_comb_last_head_body)(
        p2, inv16, hr2, W_l2, g_2, be_2, h1, h2,
        W_jk, b_jk, g_jk, be_jk, W_p1, b_p1, g_p, be_p, W_p2, b_p2)


# speedup vs baseline: 1.1433x; 1.1069x over previous
import functools

import jax
import jax.numpy as jnp
from jax import lax
from jax.experimental import pallas as pl
from jax.experimental.pallas import tpu as pltpu
from jax.experimental.pallas import tpu_sc as plsc

_N = 10000
_H = 128
_E = 320000
_NC = 2
_NS = 16
_NW = _NC * _NS
_CH = 200
_ROWS_W = 50
_NB = 10
_NBLK = _ROWS_W // _NB
_NPAD = 10112
_RPT = _NPAD // _NS

_mesh = plsc.VectorSubcoreMesh(core_axis_name="c", subcore_axis_name="s")
_sc_params = pltpu.CompilerParams(use_tc_tiling_on_sc=False)

_ZPIECES = tuple((o, min(_CH, _RPT - o)) for o in range(0, _RPT, _CH))


def _zero_rows(ref, nrows, width):
    zv = jnp.zeros((16,), jnp.float32)

    @pl.loop(0, nrows)
    def _(r):
        @pl.loop(0, width, step=16)
        def _(k):
            ref[r, pl.ds(k, 16)] = zv


def _sc_agg_body(h_hbm, srcp_hbm, dstp_hbm, out_hbm,
                 acc, srcv, dstv, rows0, rows1, rows2, rows3,
                 sem0, sem1, sem2, sem3, semi):
    c = lax.axis_index("c")
    s = lax.axis_index("s")
    wid = c * _NS + s
    bufs = (rows0, rows1, rows2, rows3)
    sems = (sem0, sem1, sem2, sem3)
    zv = jnp.zeros((32,), jnp.bfloat16)

    @pl.loop(0, _CH)
    def _(r):
        @pl.loop(0, _H, step=32)
        def _(k):
            rows0[r, pl.ds(k, 32)] = zv
    zb = s * _RPT
    zcps = [pltpu.async_copy(rows0.at[pl.ds(0, n)],
                             acc.at[pl.ds(zb + off, n)], semi)
            for off, n in _ZPIECES]
    for zcp in zcps:
        zcp.wait()
    plsc.subcore_barrier()

    base0 = wid * _ROWS_W
    pltpu.sync_copy(srcp_hbm.at[pl.ds(base0, _ROWS_W)], srcv)
    pltpu.sync_copy(dstp_hbm.at[pl.ds(base0, _ROWS_W)], dstv)
    for k in range(3):
        pltpu.async_copy(h_hbm.at[srcv.at[k]], bufs[k], sems[k])

    @pl.loop(0, _ROWS_W - 2, step=4)
    def _(j):
        for k in range(4):
            kf = (k + 3) % 4

            @pl.when(j + k + 3 < _ROWS_W)
            def _(k=k, kf=kf):
                pltpu.async_copy(h_hbm.at[srcv.at[j + k + 3]],
                                 bufs[kf], sems[kf])

            pltpu.make_async_copy(h_hbm.at[srcv.at[j + k]],
                                  bufs[k], sems[k]).wait()
            pltpu.sync_copy(bufs[k], acc.at[dstv.at[j + k]], add=True)

    for ctail in (_ROWS_W - 2, _ROWS_W - 1):
        kk = ctail % 4
        pltpu.make_async_copy(h_hbm.at[srcv.at[ctail]],
                              bufs[kk], sems[kk]).wait()
        pltpu.sync_copy(bufs[kk], acc.at[dstv.at[ctail]], add=True)

    plsc.subcore_barrier()
    pltpu.sync_copy(acc.at[pl.ds(zb, _RPT)], out_hbm.at[c, pl.ds(zb, _RPT)])


_sc_agg = pl.kernel(
    _sc_agg_body,
    out_type=jax.ShapeDtypeStruct((_NC, _NPAD, _H), jnp.bfloat16),
    mesh=_mesh,
    scratch_types=[
        pltpu.VMEM_SHARED((_NPAD, _H), jnp.bfloat16),
        pltpu.VMEM((_ROWS_W, _CH), jnp.int32),
        pltpu.VMEM((_ROWS_W, _CH), jnp.int32),
        pltpu.VMEM((_CH, _H), jnp.bfloat16),
        pltpu.VMEM((_CH, _H), jnp.bfloat16),
        pltpu.VMEM((_CH, _H), jnp.bfloat16),
        pltpu.VMEM((_CH, _H), jnp.bfloat16),
        pltpu.SemaphoreType.DMA,
        pltpu.SemaphoreType.DMA,
        pltpu.SemaphoreType.DMA,
        pltpu.SemaphoreType.DMA,
        pltpu.SemaphoreType.DMA,
    ],
    compiler_params=_sc_params,
)


def _sc_cnt_body(dstp_hbm, cnt_hbm, cnt_acc, dstv, onesv, sem0):
    c = lax.axis_index("c")
    s = lax.axis_index("s")
    wid = c * _NS + s
    _zero_rows(onesv, _CH, 16)
    zb = s * _RPT
    for off, n in _ZPIECES:
        pltpu.sync_copy(onesv.at[pl.ds(0, n)], cnt_acc.at[pl.ds(zb + off, n)])
    ov = jnp.full((16,), 1.0, jnp.float32)

    @pl.loop(0, _CH)
    def _(r):
        onesv[r, :] = ov

    plsc.subcore_barrier()

    @pl.loop(0, _NBLK)
    def _(b):
        pltpu.sync_copy(
            dstp_hbm.at[pl.ds(wid * _ROWS_W + b * _NB, _NB)], dstv)

        @pl.loop(0, _NB)
        def _(j):
            pltpu.sync_copy(onesv, cnt_acc.at[dstv.at[j]], add=True)

    plsc.subcore_barrier()
    pltpu.sync_copy(cnt_acc.at[pl.ds(zb, _RPT)], cnt_hbm.at[c, pl.ds(zb, _RPT)])


_sc_cnt = pl.kernel(
    _sc_cnt_body,
    out_type=jax.ShapeDtypeStruct((_NC, _NPAD, 16), jnp.float32),
    mesh=_mesh,
    scratch_types=[
        pltpu.VMEM_SHARED((_NPAD, 16), jnp.float32),
        pltpu.VMEM((_NB, _CH), jnp.int32),
        pltpu.VMEM((_CH, 16), jnp.float32),
        pltpu.SemaphoreType.DMA,
    ],
    compiler_params=_sc_params,
)


def _ln_relu(y, g, be):
    m = jnp.mean(y, axis=-1, keepdims=True)
    v = jnp.mean((y - m) * (y - m), axis=-1, keepdims=True)
    return jax.nn.relu((y - m) * lax.rsqrt(v + 1e-5) * g + be)


def _in_body(x_ref, w_ref, b_ref, g_ref, be_ref, wr_ref, blr_ref,
             hb_ref, hr_ref):
    y = jnp.dot(x_ref[...], w_ref[...], preferred_element_type=jnp.float32)
    h = _ln_relu(y + b_ref[...], g_ref[...], be_ref[...])
    hb_ref[...] = h.astype(jnp.bfloat16)
    hr_ref[...] = jnp.dot(h, wr_ref[...],
                          preferred_element_type=jnp.float32) + blr_ref[...]


def _combine(p_ref, inv, hr_ref, wl_ref, g_ref, be_ref):
    psum = (p_ref[0, :_N, :].astype(jnp.float32)
            + p_ref[1, :_N, :].astype(jnp.float32))
    agg = psum * inv
    out = jnp.dot(agg, wl_ref[...], preferred_element_type=jnp.float32)
    out = out + hr_ref[...]
    nrm = jnp.sqrt(jnp.sum(out * out, axis=-1, keepdims=True))
    out = out / jnp.maximum(nrm, 1e-12)
    return _ln_relu(out, g_ref[...], be_ref[...])


def _comb0_body(p_ref, cp_ref, hr_ref, wl_ref, g_ref, be_ref,
                wr_ref, blr_ref, h_ref, hb_ref, inv_ref, hrn_ref):
    cnt = cp_ref[0, :_N, :] + cp_ref[1, :_N, :]
    inv16 = 1.0 / jnp.maximum(cnt, 1.0)
    inv_ref[...] = inv16
    h = _combine(p_ref, inv16[:, 0:1], hr_ref, wl_ref, g_ref, be_ref)
    h_ref[...] = h
    hb_ref[...] = h.astype(jnp.bfloat16)
    hrn_ref[...] = jnp.dot(h, wr_ref[...],
                           preferred_element_type=jnp.float32) + blr_ref[...]


def _comb_mid_body(p_ref, inv_ref, hr_ref, wl_ref, g_ref, be_ref,
                   wr_ref, blr_ref, h_ref, hb_ref, hrn_ref):
    h = _combine(p_ref, inv_ref[:, 0:1], hr_ref, wl_ref, g_ref, be_ref)
    h_ref[...] = h
    hb_ref[...] = h.astype(jnp.bfloat16)
    hrn_ref[...] = jnp.dot(h, wr_ref[...],
                           preferred_element_type=jnp.float32) + blr_ref[...]


def _comb_last_head_body(p_ref, inv_ref, hr_ref, wl_ref, g_ref, be_ref,
                         h1_ref, h2_ref, wjk_ref, bjk_ref, gjk_ref, bejk_ref,
                         wp1_ref, bp1_ref, gp_ref, bep_ref, wp2_ref, bp2_ref,
                         o_ref):
    h3 = _combine(p_ref, inv_ref[:, 0:1], hr_ref, wl_ref, g_ref, be_ref)
    y = jnp.dot(h1_ref[...], wjk_ref[0:128, :], preferred_element_type=jnp.float32)
    y = y + jnp.dot(h2_ref[...], wjk_ref[128:256, :], preferred_element_type=jnp.float32)
    y = y + jnp.dot(h3, wjk_ref[256:384, :], preferred_element_type=jnp.float32)
    h = _ln_relu(y + bjk_ref[...], gjk_ref[...], bejk_ref[...])
    y2 = jnp.dot(h, wp1_ref[...], preferred_element_type=jnp.float32) + bp1_ref[...]
    hp = _ln_relu(y2, gp_ref[...], bep_ref[...])
    o_ref[...] = jnp.dot(hp, wp2_ref[...], preferred_element_type=jnp.float32) + bp2_ref[...]


def _tc(body, *out_specs):
    if not out_specs:
        out_specs = ((_H, jnp.float32),)
    shapes = tuple(jax.ShapeDtypeStruct((_N, c), dt) for c, dt in out_specs)
    return pl.pallas_call(
        body, out_shape=shapes if len(shapes) > 1 else shapes[0])


def kernel(x, edge_index, W_in, b_in, g_in, be_in, W_l0, b_l0, W_r0, g_0, be_0,
           W_l1, b_l1, W_r1, g_1, be_1, W_l2, b_l2, W_r2, g_2, be_2,
           W_jk, b_jk, g_jk, be_jk, W_p1, b_p1, g_p, be_p, W_p2, b_p2):
    srcp = edge_index[0].reshape(_E // _CH, _CH)
    dstp = edge_index[1].reshape(_E // _CH, _CH)

    cp = _sc_cnt(dstp)
    h0b, hr0 = _tc(_in_body, (_H, jnp.bfloat16), (_H, jnp.float32))(
        x, W_in, b_in, g_in, be_in, W_r0, b_l0)

    p0 = _sc_agg(h0b, srcp, dstp)
    h1, h1b, inv16, hr1 = _tc(
        _comb0_body, (_H, jnp.float32), (_H, jnp.bfloat16),
        (16, jnp.float32), (_H, jnp.float32))(
        p0, cp, hr0, W_l0, g_0, be_0, W_r1, b_l1)

    p1 = _sc_agg(h1b, srcp, dstp)
    h2, h2b, hr2 = _tc(
        _comb_mid_body, (_H, jnp.float32), (_H, jnp.bfloat16),
        (_H, jnp.float32))(
        p1, inv16, hr1, W_l1, g_1, be_1, W_r2, b_l2)

    p2 = _sc_agg(h2b, srcp, dstp)
    return _tc(_comb_last_head_body)(
        p2, inv16, hr2, W_l2, g_2, be_2, h1, h2,
        W_jk, b_jk, g_jk, be_jk, W_p1, b_p1, g_p, be_p, W_p2, b_p2)
